# baseline (speedup 1.0000x reference)
import functools

import jax
import jax.numpy as jnp
from jax import lax
from jax.experimental import pallas as pl
from jax.experimental.pallas import tpu as pltpu
from jax.experimental.pallas import tpu_sc as plsc

_NUM_CORES = 2
_NUM_SUBCORES = 16
_NUM_WORKERS = _NUM_CORES * _NUM_SUBCORES
_CHUNK = 128


def kernel(token_ids, weight):
    batch, seq = token_ids.shape
    n = batch * seq
    dim = weight.shape[1]
    idx = token_ids.reshape(n).astype(jnp.int32)

    per_worker = n // _NUM_WORKERS
    steps = per_worker // _CHUNK

    mesh = plsc.VectorSubcoreMesh(core_axis_name="c", subcore_axis_name="s")

    @functools.partial(
        pl.kernel,
        mesh=mesh,
        out_type=jax.ShapeDtypeStruct((n, dim), jnp.float32),
        scratch_types=[
            pltpu.VMEM((_CHUNK,), jnp.int32),
            pltpu.VMEM((_CHUNK, dim), jnp.float32),
            pltpu.SemaphoreType.DMA,
        ],
        compiler_params=pltpu.CompilerParams(use_tc_tiling_on_sc=False),
    )
    def gather_kernel(table_hbm, idx_hbm, out_hbm, idx_v, rows_v, sem):
        wid = lax.axis_index("s") * _NUM_CORES + lax.axis_index("c")
        base = wid * per_worker

        @pl.loop(0, steps)
        def _(i):
            off = base + i * _CHUNK
            pltpu.sync_copy(idx_hbm.at[pl.ds(off, _CHUNK)], idx_v)
            pltpu.async_copy(table_hbm.at[idx_v], rows_v, sem).wait()
            pltpu.sync_copy(rows_v, out_hbm.at[pl.ds(off, _CHUNK)])

    out = gather_kernel(weight, idx)
    return out.reshape(batch, seq, dim)


# speedup vs baseline: 1.0744x; 1.0744x over previous
import functools

import jax
import jax.numpy as jnp
from jax import lax
from jax.experimental import pallas as pl
from jax.experimental.pallas import tpu as pltpu
from jax.experimental.pallas import tpu_sc as plsc

_NUM_CORES = 2
_NUM_SUBCORES = 16
_NUM_WORKERS = _NUM_CORES * _NUM_SUBCORES
_CHUNK = 128
_INFLIGHT = 5
_NSLOT = 10


def kernel(token_ids, weight):
    batch, seq = token_ids.shape
    n = batch * seq
    dim = weight.shape[1]

    per_worker = n // _NUM_WORKERS
    steps = per_worker // _CHUNK
    idx = token_ids.reshape(_NUM_WORKERS, steps, _CHUNK).astype(jnp.int32)

    mesh = plsc.VectorSubcoreMesh(core_axis_name="c", subcore_axis_name="s")

    @functools.partial(
        pl.kernel,
        mesh=mesh,
        out_type=jax.ShapeDtypeStruct((n, dim), jnp.float32),
        scratch_types=[
            pltpu.VMEM((steps, _CHUNK), jnp.int32),
            pltpu.VMEM((_NSLOT, _CHUNK, dim), jnp.float32),
            pltpu.SemaphoreType.DMA((_NSLOT,)),
            pltpu.SemaphoreType.DMA((_NSLOT,)),
        ],
        compiler_params=pltpu.CompilerParams(use_tc_tiling_on_sc=False),
    )
    def gather_kernel(table_hbm, idx_hbm, out_hbm, idx_v, rows_v, gsem, osem):
        wid = lax.axis_index("s") * _NUM_CORES + lax.axis_index("c")
        base = wid * per_worker
        pltpu.sync_copy(idx_hbm.at[wid], idx_v)

        gather_d = {}
        out_d = {}

        def start_gather(j):
            slot = j % _NSLOT
            gather_d[j] = pltpu.async_copy(
                table_hbm.at[idx_v.at[j]], rows_v.at[slot], gsem.at[slot]
            )

        def start_out(i):
            slot = i % _NSLOT
            out_d[i] = pltpu.async_copy(
                rows_v.at[slot],
                out_hbm.at[pl.ds(base + i * _CHUNK, _CHUNK)],
                osem.at[slot],
            )

        for j in range(_INFLIGHT):
            start_gather(j)
        for i in range(steps):
            j = i + _INFLIGHT
            if j < steps:
                if j >= _NSLOT:
                    out_d[j - _NSLOT].wait()
                start_gather(j)
            gather_d[i].wait()
            start_out(i)
        for k in range(max(0, steps - _NSLOT), steps):
            out_d[k].wait()

    out = gather_kernel(weight, idx)
    return out.reshape(batch, seq, dim)
